# baseline (speedup 1.0000x reference)
import functools

import jax
import jax.numpy as jnp
from jax import lax
from jax.experimental import pallas as pl
from jax.experimental.pallas import tpu as pltpu
from jax.experimental.pallas import tpu_sc as plsc

N = 1000000
LANES = 16
NSUB = 16
BLK = 8000
NBLK = N // BLK
VEC_PER_BLK = BLK // LANES

_MESH = plsc.VectorSubcoreMesh(
    core_axis_name="c", subcore_axis_name="s", num_cores=1, num_subcores=NSUB
)


def _body(score_hbm, out_hbm, buf, pvec_f, pvec_i, shared_f, shared_i,
          comb_f, comb_i, out_stage):
    wid = lax.axis_index("s")

    def block_body(k, carry):
        s_acc, c_acc = carry
        bid = wid + NSUB * k
        pltpu.sync_copy(score_hbm.at[pl.ds(bid * BLK, BLK)], buf)

        def vec_body(i, carry2):
            s, c = carry2
            v = buf[pl.ds(i * LANES, LANES)]
            s = s + jnp.minimum(v, 0.0)
            c = c + (plsc.bitcast(v, jnp.int32) >> 31)
            return s, c

        return lax.fori_loop(0, VEC_PER_BLK, vec_body, (s_acc, c_acc))

    nb = jnp.where(wid < NBLK - 7 * NSUB, 8, 7)
    s0 = jnp.zeros((LANES,), jnp.float32)
    c0 = jnp.zeros((LANES,), jnp.int32)
    s_fin, c_fin = lax.fori_loop(0, nb, block_body, (s0, c0))

    pvec_f[...] = s_fin
    pvec_i[...] = c_fin
    pltpu.sync_copy(pvec_f, shared_f.at[wid])
    pltpu.sync_copy(pvec_i, shared_i.at[wid])
    plsc.subcore_barrier()

    @pl.when(wid == 0)
    def _():
        pltpu.sync_copy(shared_f, comb_f)
        pltpu.sync_copy(shared_i, comb_i)
        s_vec = comb_f[0, :]
        c_vec = comb_i[0, :]
        for i in range(1, NSUB):
            s_vec = s_vec + comb_f[i, :]
            c_vec = c_vec + comb_i[i, :]
        total_s = jnp.sum(s_vec)
        total_c = (-jnp.sum(c_vec)).astype(jnp.float32)
        num = jnp.broadcast_to(total_s, (LANES,))
        den = jnp.broadcast_to(total_c, (LANES,))
        out_stage[...] = num / den
        pltpu.sync_copy(out_stage, out_hbm)


_margin_call = functools.partial(
    pl.kernel,
    out_type=jax.ShapeDtypeStruct((LANES,), jnp.float32),
    mesh=_MESH,
    compiler_params=pltpu.CompilerParams(needs_layout_passes=False),
    scratch_types=[
        pltpu.VMEM((BLK,), jnp.float32),
        pltpu.VMEM((LANES,), jnp.float32),
        pltpu.VMEM((LANES,), jnp.int32),
        pltpu.VMEM_SHARED((NSUB, LANES), jnp.float32),
        pltpu.VMEM_SHARED((NSUB, LANES), jnp.int32),
        pltpu.VMEM((NSUB, LANES), jnp.float32),
        pltpu.VMEM((NSUB, LANES), jnp.int32),
        pltpu.VMEM((LANES,), jnp.float32),
    ],
)(_body)


def kernel(preds, lables, score):
    del preds, lables
    return _margin_call(score)[0]


# speedup vs baseline: 1.6819x; 1.6819x over previous
import functools

import jax
import jax.numpy as jnp
from jax import lax
from jax.experimental import pallas as pl
from jax.experimental.pallas import tpu as pltpu
from jax.experimental.pallas import tpu_sc as plsc

N = 1000000
LANES = 16
NSUB = 16
NSBLK = 4
SUB = 15616
W = NSBLK * SUB
VPS = SUB // LANES
UNROLL = 8
ITERS = VPS // UNROLL
TAIL = N - NSUB * W
TAIL_OFF = NSUB * W
TAIL_VECS = TAIL // LANES

_MESH = plsc.VectorSubcoreMesh(
    core_axis_name="c", subcore_axis_name="s", num_cores=1, num_subcores=NSUB
)


def _neg_update(v, s, c):
    s = s + jnp.minimum(v, 0.0)
    c = c + (plsc.bitcast(v, jnp.int32) >> 31)
    return s, c


def _body(score_hbm, out_hbm, buf, tbuf, pvec_f, pvec_i, shared_f, shared_i,
          comb_f, comb_i, out_stage, sems):
    wid = lax.axis_index("s")
    base = wid * W

    copies = [
        pltpu.async_copy(
            score_hbm.at[pl.ds(base + b * SUB, SUB)], buf.at[b], sems.at[b]
        )
        for b in range(NSBLK)
    ]

    zf = jnp.zeros((LANES,), jnp.float32)
    zi = jnp.zeros((LANES,), jnp.int32)
    ss = [zf] * 4
    cc = [zi] * 4

    for b in range(NSBLK):
        copies[b].wait()

        def vec_body(t, carry, _b=b):
            (s0, s1, s2, s3), (c0, c1, c2, c3) = carry
            sl = [s0, s1, s2, s3]
            cl = [c0, c1, c2, c3]
            off = t * (UNROLL * LANES)
            for j in range(UNROLL):
                v = buf[_b, pl.ds(off + j * LANES, LANES)]
                k = j % 4
                sl[k], cl[k] = _neg_update(v, sl[k], cl[k])
            return tuple(sl), tuple(cl)

        ss, cc = lax.fori_loop(0, ITERS, vec_body, (tuple(ss), tuple(cc)))
        ss, cc = list(ss), list(cc)

    s_fin = (ss[0] + ss[1]) + (ss[2] + ss[3])
    c_fin = (cc[0] + cc[1]) + (cc[2] + cc[3])
    pvec_f[...] = s_fin
    pvec_i[...] = c_fin

    @pl.when(wid == 0)
    def _tail():
        pltpu.sync_copy(score_hbm.at[pl.ds(TAIL_OFF, TAIL)], tbuf)
        st, ct = pvec_f[...], pvec_i[...]
        for j in range(TAIL_VECS):
            v = tbuf[pl.ds(j * LANES, LANES)]
            st, ct = _neg_update(v, st, ct)
        pvec_f[...] = st
        pvec_i[...] = ct

    pltpu.sync_copy(pvec_f, shared_f.at[wid])
    pltpu.sync_copy(pvec_i, shared_i.at[wid])
    plsc.subcore_barrier()

    @pl.when(wid == 0)
    def _combine():
        pltpu.sync_copy(shared_f, comb_f)
        pltpu.sync_copy(shared_i, comb_i)
        s_vec = comb_f[0, :]
        c_vec = comb_i[0, :]
        for i in range(1, NSUB):
            s_vec = s_vec + comb_f[i, :]
            c_vec = c_vec + comb_i[i, :]
        total_s = jnp.sum(s_vec)
        total_c = (-jnp.sum(c_vec)).astype(jnp.float32)
        num = jnp.broadcast_to(total_s, (LANES,))
        den = jnp.broadcast_to(total_c, (LANES,))
        out_stage[...] = num / den
        pltpu.sync_copy(out_stage, out_hbm)


_margin_call = functools.partial(
    pl.kernel,
    out_type=jax.ShapeDtypeStruct((LANES,), jnp.float32),
    mesh=_MESH,
    compiler_params=pltpu.CompilerParams(needs_layout_passes=False),
    scratch_types=[
        pltpu.VMEM((NSBLK, SUB), jnp.float32),
        pltpu.VMEM((TAIL,), jnp.float32),
        pltpu.VMEM((LANES,), jnp.float32),
        pltpu.VMEM((LANES,), jnp.int32),
        pltpu.VMEM_SHARED((NSUB, LANES), jnp.float32),
        pltpu.VMEM_SHARED((NSUB, LANES), jnp.int32),
        pltpu.VMEM((NSUB, LANES), jnp.float32),
        pltpu.VMEM((NSUB, LANES), jnp.int32),
        pltpu.VMEM((LANES,), jnp.float32),
        pltpu.SemaphoreType.DMA((NSBLK,)),
    ],
)(_body)


def kernel(preds, lables, score):
    del preds, lables
    return _margin_call(score)[0]


# speedup vs baseline: 1.6982x; 1.0097x over previous
import functools

import jax
import jax.numpy as jnp
from jax import lax
from jax.experimental import pallas as pl
from jax.experimental.pallas import tpu as pltpu
from jax.experimental.pallas import tpu_sc as plsc

N = 1000000
LANES = 16
NSUB = 16
NSBLK = 2
SUB = 31232
W = NSBLK * SUB
VPS = SUB // LANES
UNROLL = 8
ITERS = VPS // UNROLL
TAIL = N - NSUB * W
TAIL_OFF = NSUB * W
TAIL_VECS = TAIL // LANES

_MESH = plsc.VectorSubcoreMesh(
    core_axis_name="c", subcore_axis_name="s", num_cores=1, num_subcores=NSUB
)


def _neg_update(v, s, c):
    s = s + jnp.minimum(v, 0.0)
    c = c + (plsc.bitcast(v, jnp.int32) >> 31)
    return s, c


def _body(score_hbm, out_hbm, buf, tbuf, pvec_f, pvec_i, shared_f, shared_i,
          comb_f, comb_i, out_stage, sems):
    wid = lax.axis_index("s")
    base = wid * W

    copies = [
        pltpu.async_copy(
            score_hbm.at[pl.ds(base + b * SUB, SUB)], buf.at[b], sems.at[b]
        )
        for b in range(NSBLK)
    ]

    zf = jnp.zeros((LANES,), jnp.float32)
    zi = jnp.zeros((LANES,), jnp.int32)
    ss = [zf] * 3
    cc = [zi] * 3

    for b in range(NSBLK):
        copies[b].wait()

        def vec_body(t, carry, _b=b):
            (s0, s1, s2), (c0, c1, c2) = carry
            sl = [s0, s1, s2]
            cl = [c0, c1, c2]
            off = t * (UNROLL * LANES)
            for j in range(UNROLL):
                v = buf[_b, pl.ds(off + j * LANES, LANES)]
                k = j % 3
                sl[k], cl[k] = _neg_update(v, sl[k], cl[k])
            return tuple(sl), tuple(cl)

        ss, cc = lax.fori_loop(0, ITERS, vec_body, (tuple(ss), tuple(cc)))
        ss, cc = list(ss), list(cc)

    s_fin = ss[0] + ss[1] + ss[2]
    c_fin = cc[0] + cc[1] + cc[2]

    @pl.when(wid == 0)
    def _tail():
        pltpu.sync_copy(score_hbm.at[pl.ds(TAIL_OFF, TAIL)], tbuf)
        st, ct = s_fin, c_fin
        for j in range(TAIL_VECS):
            v = tbuf[pl.ds(j * LANES, LANES)]
            st, ct = _neg_update(v, st, ct)
        pvec_f[...] = st
        pvec_i[...] = ct

    @pl.when(wid != 0)
    def _main_store():
        pvec_f[...] = s_fin
        pvec_i[...] = c_fin

    pltpu.sync_copy(pvec_f, shared_f.at[wid])
    pltpu.sync_copy(pvec_i, shared_i.at[wid])
    plsc.subcore_barrier()

    @pl.when(wid == 0)
    def _combine():
        pltpu.sync_copy(shared_f, comb_f)
        pltpu.sync_copy(shared_i, comb_i)
        s_vec = comb_f[0, :]
        c_vec = comb_i[0, :]
        for i in range(1, NSUB):
            s_vec = s_vec + comb_f[i, :]
            c_vec = c_vec + comb_i[i, :]
        total_s = jnp.sum(s_vec)
        total_c = (-jnp.sum(c_vec)).astype(jnp.float32)
        num = jnp.broadcast_to(total_s, (LANES,))
        den = jnp.broadcast_to(total_c, (LANES,))
        out_stage[...] = num / den
        pltpu.sync_copy(out_stage, out_hbm)


_margin_call = functools.partial(
    pl.kernel,
    out_type=jax.ShapeDtypeStruct((LANES,), jnp.float32),
    mesh=_MESH,
    compiler_params=pltpu.CompilerParams(needs_layout_passes=False),
    scratch_types=[
        pltpu.VMEM((NSBLK, SUB), jnp.float32),
        pltpu.VMEM((TAIL,), jnp.float32),
        pltpu.VMEM((LANES,), jnp.float32),
        pltpu.VMEM((LANES,), jnp.int32),
        pltpu.VMEM_SHARED((NSUB, LANES), jnp.float32),
        pltpu.VMEM_SHARED((NSUB, LANES), jnp.int32),
        pltpu.VMEM((NSUB, LANES), jnp.float32),
        pltpu.VMEM((NSUB, LANES), jnp.int32),
        pltpu.VMEM((LANES,), jnp.float32),
        pltpu.SemaphoreType.DMA((NSBLK,)),
    ],
)(_body)


def kernel(preds, lables, score):
    del preds, lables
    return _margin_call(score)[0]
